# baseline (speedup 1.0000x reference)
import functools

import jax
import jax.numpy as jnp
from jax import lax
from jax.experimental import pallas as pl
from jax.experimental.pallas import tpu as pltpu
from jax.experimental.pallas import tpu_sc as plsc

N_XCOND = 64
D = 128
BATCH = 16384

_NC = 2
_NS = 16
_NW = _NC * _NS
_BPW = BATCH // _NW
_CHUNK = 128
_NCHUNK = _BPW // _CHUNK
_L = 16


@functools.partial(
    pl.kernel,
    out_type=jax.ShapeDtypeStruct((BATCH, D), jnp.float32),
    mesh=plsc.VectorSubcoreMesh(core_axis_name="c", subcore_axis_name="s"),
    scratch_types=[
        pltpu.VMEM((_BPW,), jnp.int32),
        pltpu.VMEM((_BPW,), jnp.int32),
        pltpu.VMEM((_BPW,), jnp.int32),
        pltpu.VMEM((_BPW, D), jnp.float32),
        pltpu.SemaphoreType.DMA,
        pltpu.SemaphoreType.DMA,
    ],
)
def _embed_gather(cls_hbm, xc_hbm, table_hbm, out_hbm, cls_v, xc_v, idx_v,
                  rows_v, gsem, ssem):
    wid = lax.axis_index("s") * _NC + lax.axis_index("c")
    base = wid * _BPW

    lc = pltpu.async_copy(cls_hbm.at[pl.ds(base, _BPW)], cls_v, gsem)
    lx = pltpu.async_copy(xc_hbm.at[pl.ds(base, _BPW)], xc_v, gsem)
    lc.wait()
    lx.wait()

    for i in range(_BPW // _L):
        sl = pl.ds(i * _L, _L)
        idx_v[sl] = cls_v[sl] * N_XCOND + xc_v[sl]

    h = _BPW // 2
    g0 = pltpu.async_copy(
        table_hbm.at[idx_v.at[pl.ds(0, h)]], rows_v.at[pl.ds(0, h)], gsem)
    g1 = pltpu.async_copy(
        table_hbm.at[idx_v.at[pl.ds(h, h)]], rows_v.at[pl.ds(h, h)], gsem)
    g0.wait()
    s0 = pltpu.async_copy(
        rows_v.at[pl.ds(0, h)], out_hbm.at[pl.ds(base, h)], ssem)
    g1.wait()
    s1 = pltpu.async_copy(
        rows_v.at[pl.ds(h, h)], out_hbm.at[pl.ds(base + h, h)], ssem)
    s0.wait()
    s1.wait()


def kernel(class_ws, x_cond_ws, embedding_table):
    return _embed_gather(class_ws, x_cond_ws, embedding_table)


# speedup vs baseline: 1.0224x; 1.0224x over previous
import functools

import jax
import jax.numpy as jnp
from jax import lax
from jax.experimental import pallas as pl
from jax.experimental.pallas import tpu as pltpu
from jax.experimental.pallas import tpu_sc as plsc

N_XCOND = 64
D = 128
BATCH = 16384

_NC = 2
_NS = 16
_NW = _NC * _NS
_BPW = BATCH // _NW
_CHUNK = 128
_NCHUNK = _BPW // _CHUNK
_L = 16


@functools.partial(
    pl.kernel,
    out_type=jax.ShapeDtypeStruct((BATCH, D), jnp.float32),
    mesh=plsc.VectorSubcoreMesh(core_axis_name="c", subcore_axis_name="s"),
    scratch_types=[
        pltpu.VMEM((_BPW,), jnp.int32),
        pltpu.VMEM((_BPW,), jnp.int32),
        pltpu.VMEM((_BPW,), jnp.int32),
        pltpu.VMEM((_BPW, D), jnp.float32),
        pltpu.SemaphoreType.DMA,
        pltpu.SemaphoreType.DMA,
    ],
)
def _embed_gather(cls_hbm, xc_hbm, table_hbm, out_hbm, cls_v, xc_v, idx_v,
                  rows_v, gsem, ssem):
    wid = lax.axis_index("s") * _NC + lax.axis_index("c")
    base = wid * _BPW

    lc = pltpu.async_copy(cls_hbm.at[pl.ds(base, _BPW)], cls_v, gsem)
    lx = pltpu.async_copy(xc_hbm.at[pl.ds(base, _BPW)], xc_v, gsem)
    lc.wait()
    lx.wait()

    def _cbody(i, carry):
        sl = pl.ds(i * _L, _L)
        idx_v[sl] = cls_v[sl] * N_XCOND + xc_v[sl]
        return carry

    lax.fori_loop(0, _BPW // _L, _cbody, 0)

    pltpu.async_copy(table_hbm.at[idx_v], rows_v, gsem).wait()
    pltpu.async_copy(rows_v, out_hbm.at[pl.ds(base, _BPW)], ssem).wait()


def kernel(class_ws, x_cond_ws, embedding_table):
    return _embed_gather(class_ws, x_cond_ws, embedding_table)


# speedup vs baseline: 1.0284x; 1.0059x over previous
import functools

import jax
import jax.numpy as jnp
from jax import lax
from jax.experimental import pallas as pl
from jax.experimental.pallas import tpu as pltpu
from jax.experimental.pallas import tpu_sc as plsc

N_XCOND = 64
D = 128
BATCH = 16384

_NC = 2
_NS = 16
_NW = _NC * _NS
_BPW = BATCH // _NW
_CHUNK = 128
_NCHUNK = _BPW // _CHUNK
_L = 16


@functools.partial(
    pl.kernel,
    out_type=jax.ShapeDtypeStruct((BATCH, D), jnp.float32),
    mesh=plsc.VectorSubcoreMesh(core_axis_name="c", subcore_axis_name="s"),
    scratch_types=[
        pltpu.VMEM((_BPW,), jnp.int32),
        pltpu.VMEM((_BPW,), jnp.int32),
        pltpu.VMEM((_BPW,), jnp.int32),
        pltpu.VMEM((_BPW, D), jnp.float32),
        pltpu.SemaphoreType.DMA,
        pltpu.SemaphoreType.DMA,
    ],
)
def _embed_gather(cls_hbm, xc_hbm, table_hbm, out_hbm, cls_v, xc_v, idx_v,
                  rows_v, gsem, ssem):
    wid = lax.axis_index("s") * _NC + lax.axis_index("c")
    base = wid * _BPW

    lc = pltpu.async_copy(cls_hbm.at[pl.ds(base, _BPW)], cls_v, gsem)
    lx = pltpu.async_copy(xc_hbm.at[pl.ds(base, _BPW)], xc_v, gsem)
    lc.wait()
    lx.wait()

    for i in range(_BPW // _L):
        sl = pl.ds(i * _L, _L)
        idx_v[sl] = cls_v[sl] * N_XCOND + xc_v[sl]

    pltpu.async_copy(table_hbm.at[idx_v], rows_v, gsem).wait()
    pltpu.async_copy(rows_v, out_hbm.at[pl.ds(base, _BPW)], ssem).wait()


def kernel(class_ws, x_cond_ws, embedding_table):
    return _embed_gather(class_ws, x_cond_ws, embedding_table)
